# baseline (speedup 1.0000x reference)
import jax
import jax.numpy as jnp
from jax import lax
from jax.experimental import pallas as pl
from jax.experimental.pallas import tpu as pltpu
from jax.experimental.pallas import tpu_sc as plsc

N_ELEMS = 1_600_000
N_SEG = 100_000
SEG_PAD = 100_352
NC = 2
NS = 16
NW = NC * NS
LANES = 128
CHUNK_E = 8_320
NCHUNK = 6
PER_TILE = CHUNK_E * NCHUNK
EPI_BASE = NW * PER_TILE
NSLOT = 3
SLICE = SEG_PAD // NS

_LN2 = 0.69314718
_NEG_BIG = -1.0e30
_Q0 = 0.9999670988417516
_Q1 = -0.4994411088193433
_Q2 = 0.33632475570351283
_Q3 = -0.2711059246189344
_Q4 = 0.17721477123404433
_SQRT2M1_BITS = 0x3504F3


def _log16(xv):
    bits = lax.bitcast_convert_type(xv, jnp.int32)
    eb = ((bits - _SQRT2M1_BITS) >> 23) - 126
    m = lax.bitcast_convert_type(bits - (eb << 23), jnp.float32)
    zz = m - 1.0
    q = _Q4
    q = q * zz + _Q3
    q = q * zz + _Q2
    q = q * zz + _Q1
    q = q * zz + _Q0
    logx = eb.astype(jnp.float32) * _LN2 + q * zz
    return jnp.where(xv < 1.1754944e-38, _NEG_BIG, logx)


def _log_flat(buf, n):
    @plsc.parallel_loop(0, n // 128, step=1, unroll=2)
    def vloop(v):
        for k in range(8):
            sl = pl.ds(v * 128 + k * 16, 16)
            buf[sl] = _log16(buf[sl])


def _sc_body(x_hbm, y_hbm, part_hbm, xb, yb, zbuf, xe, ye, acc,
             sin0, sin1, sin2, ssc0, ssc1, ssc2):
    cid = lax.axis_index("c")
    sid = lax.axis_index("s")
    wid = sid * NC + cid
    base = wid * PER_TILE
    s_in = (sin0, sin1, sin2)
    s_sc = (ssc0, ssc1, ssc2)

    pltpu.async_copy(x_hbm.at[pl.ds(base, CHUNK_E)], xb.at[0, 0], sin0)
    pltpu.async_copy(y_hbm.at[pl.ds(base, CHUNK_E)], yb.at[0, 0], sin0)

    zero = jnp.zeros((16,), jnp.float32)

    @plsc.parallel_loop(0, SLICE // 16, step=1, unroll=8)
    def zloop(i):
        zbuf[pl.ds(i * 16, 16)] = zero

    pltpu.sync_copy(zbuf, acc.at[pl.ds(sid * SLICE, SLICE)])
    plsc.subcore_barrier()

    def chunk_step(n, b):
        nb = (b + 1) % NSLOT

        @pl.when(n >= 2)
        def _():
            pltpu.make_async_copy(
                xb.at[nb, 0], acc.at[yb.at[nb, 0]], s_sc[nb]).wait()

        @pl.when(n + 1 < NCHUNK)
        def _():
            noff = base + (n + 1) * CHUNK_E
            pltpu.async_copy(
                x_hbm.at[pl.ds(noff, CHUNK_E)], xb.at[nb, 0], s_in[nb])
            pltpu.async_copy(
                y_hbm.at[pl.ds(noff, CHUNK_E)], yb.at[nb, 0], s_in[nb])

        off = base + n * CHUNK_E
        pltpu.make_async_copy(
            x_hbm.at[pl.ds(off, CHUNK_E)], xb.at[b, 0], s_in[b]).wait()
        pltpu.make_async_copy(
            y_hbm.at[pl.ds(off, CHUNK_E)], yb.at[b, 0], s_in[b]).wait()

        _log_flat(xb.at[b, 0], CHUNK_E)

        pltpu.async_copy(xb.at[b, 0], acc.at[yb.at[b, 0]], s_sc[b], add=True)

    def chunk_trip(g, carry):
        for b in range(NSLOT):
            chunk_step(g * NSLOT + b, b)
        return carry

    lax.fori_loop(0, NCHUNK // NSLOT, chunk_trip, 0)

    for s in (1, 2):
        pltpu.make_async_copy(
            xb.at[s, 0], acc.at[yb.at[s, 0]], s_sc[s]).wait()

    @pl.when(wid < (N_ELEMS - EPI_BASE) // LANES)
    def _():
        off_e = EPI_BASE + wid * LANES
        pltpu.sync_copy(x_hbm.at[pl.ds(off_e, LANES)], xe.at[0])
        pltpu.sync_copy(y_hbm.at[pl.ds(off_e, LANES)], ye.at[0])
        for k in range(LANES // 16):
            sl = pl.ds(k * 16, 16)
            xe[0, sl] = _log16(xe[0, sl])
        pltpu.sync_copy(xe.at[0], acc.at[ye.at[0]], add=True)

    plsc.subcore_barrier()

    pltpu.sync_copy(acc.at[pl.ds(sid * SLICE, SLICE)],
                    part_hbm.at[cid, pl.ds(sid * SLICE, SLICE)])


def _combine_body(p_ref, o_ref):
    o_ref[...] = jnp.exp(p_ref[0, :] + p_ref[1, :])


@jax.jit
def _segment_prod(x, y):
    mesh = plsc.VectorSubcoreMesh(core_axis_name="c", subcore_axis_name="s")
    partials = pl.kernel(
        _sc_body,
        out_type=jax.ShapeDtypeStruct((NC, SEG_PAD), jnp.float32),
        mesh=mesh,
        scratch_types=[
            pltpu.VMEM((NSLOT, 1, CHUNK_E), jnp.float32),
            pltpu.VMEM((NSLOT, 1, CHUNK_E), jnp.int32),
            pltpu.VMEM((SLICE,), jnp.float32),
            pltpu.VMEM((1, LANES), jnp.float32),
            pltpu.VMEM((1, LANES), jnp.int32),
            pltpu.VMEM_SHARED((SEG_PAD,), jnp.float32),
            pltpu.SemaphoreType.DMA,
            pltpu.SemaphoreType.DMA,
            pltpu.SemaphoreType.DMA,
            pltpu.SemaphoreType.DMA,
            pltpu.SemaphoreType.DMA,
            pltpu.SemaphoreType.DMA,
        ],
        compiler_params=pltpu.CompilerParams(needs_layout_passes=False),
    )(x, y)

    combined = pl.pallas_call(
        _combine_body,
        in_specs=[pl.BlockSpec((NC, SEG_PAD), lambda: (0, 0))],
        out_specs=pl.BlockSpec((SEG_PAD,), lambda: (0,)),
        out_shape=jax.ShapeDtypeStruct((SEG_PAD,), jnp.float32),
    )(partials)
    return combined[:N_SEG]


def kernel(x, y, z):
    del z
    return _segment_prod(x, y)


# speedup vs baseline: 1.0612x; 1.0612x over previous
import jax
import jax.numpy as jnp
from jax import lax
from jax.experimental import pallas as pl
from jax.experimental.pallas import tpu as pltpu
from jax.experimental.pallas import tpu_sc as plsc

N_ELEMS = 1_600_000
N_SEG = 100_000
SEG_PAD = 100_352
NC = 2
NS = 16
NW = NC * NS
LANES = 128
CHUNK_E = 3_328
NCHUNK = 15
PER_TILE = CHUNK_E * NCHUNK
EPI_BASE = NW * PER_TILE
NSLOT = 3
SLICE = SEG_PAD // NS

_LN2 = 0.69314718
_NEG_BIG = -1.0e30
_Q0 = 0.9999670988417516
_Q1 = -0.4994411088193433
_Q2 = 0.33632475570351283
_Q3 = -0.2711059246189344
_Q4 = 0.17721477123404433
_SQRT2M1_BITS = 0x3504F3


def _log16(xv):
    bits = lax.bitcast_convert_type(xv, jnp.int32)
    eb = ((bits - _SQRT2M1_BITS) >> 23) - 126
    m = lax.bitcast_convert_type(bits - (eb << 23), jnp.float32)
    zz = m - 1.0
    q = _Q4
    q = q * zz + _Q3
    q = q * zz + _Q2
    q = q * zz + _Q1
    q = q * zz + _Q0
    logx = eb.astype(jnp.float32) * _LN2 + q * zz
    return jnp.where(xv < 1.1754944e-38, _NEG_BIG, logx)


def _log_flat(buf, n):
    @plsc.parallel_loop(0, n // 256, step=1, unroll=1)
    def vloop(v):
        for k in range(16):
            sl = pl.ds(v * 256 + k * 16, 16)
            buf[sl] = _log16(buf[sl])


def _sc_body(x_hbm, y_hbm, part_hbm, xb, yb, zbuf, xe, ye, acc,
             sin0, sin1, sin2, ssc0, ssc1, ssc2):
    cid = lax.axis_index("c")
    sid = lax.axis_index("s")
    wid = sid * NC + cid
    base = wid * PER_TILE
    s_in = (sin0, sin1, sin2)
    s_sc = (ssc0, ssc1, ssc2)

    pltpu.async_copy(x_hbm.at[pl.ds(base, CHUNK_E)], xb.at[0, 0], sin0)
    pltpu.async_copy(y_hbm.at[pl.ds(base, CHUNK_E)], yb.at[0, 0], sin0)

    zero = jnp.zeros((16,), jnp.float32)

    @plsc.parallel_loop(0, SLICE // 16, step=1, unroll=8)
    def zloop(i):
        zbuf[pl.ds(i * 16, 16)] = zero

    pltpu.sync_copy(zbuf, acc.at[pl.ds(sid * SLICE, SLICE)])
    plsc.subcore_barrier()

    def chunk_step(n, b):
        nb = (b + 1) % NSLOT

        @pl.when(n >= 2)
        def _():
            pltpu.make_async_copy(
                xb.at[nb, 0], acc.at[yb.at[nb, 0]], s_sc[nb]).wait()

        @pl.when(n + 1 < NCHUNK)
        def _():
            noff = base + (n + 1) * CHUNK_E
            pltpu.async_copy(
                x_hbm.at[pl.ds(noff, CHUNK_E)], xb.at[nb, 0], s_in[nb])
            pltpu.async_copy(
                y_hbm.at[pl.ds(noff, CHUNK_E)], yb.at[nb, 0], s_in[nb])

        off = base + n * CHUNK_E
        pltpu.make_async_copy(
            x_hbm.at[pl.ds(off, CHUNK_E)], xb.at[b, 0], s_in[b]).wait()
        pltpu.make_async_copy(
            y_hbm.at[pl.ds(off, CHUNK_E)], yb.at[b, 0], s_in[b]).wait()

        _log_flat(xb.at[b, 0], CHUNK_E)

        pltpu.async_copy(xb.at[b, 0], acc.at[yb.at[b, 0]], s_sc[b], add=True)

    def chunk_trip(g, carry):
        for b in range(NSLOT):
            chunk_step(g * NSLOT + b, b)
        return carry

    lax.fori_loop(0, NCHUNK // NSLOT, chunk_trip, 0)

    for s in (1, 2):
        pltpu.make_async_copy(
            xb.at[s, 0], acc.at[yb.at[s, 0]], s_sc[s]).wait()

    @pl.when(wid < (N_ELEMS - EPI_BASE) // LANES)
    def _():
        off_e = EPI_BASE + wid * LANES
        pltpu.sync_copy(x_hbm.at[pl.ds(off_e, LANES)], xe.at[0])
        pltpu.sync_copy(y_hbm.at[pl.ds(off_e, LANES)], ye.at[0])
        for k in range(LANES // 16):
            sl = pl.ds(k * 16, 16)
            xe[0, sl] = _log16(xe[0, sl])
        pltpu.sync_copy(xe.at[0], acc.at[ye.at[0]], add=True)

    plsc.subcore_barrier()

    pltpu.sync_copy(acc.at[pl.ds(sid * SLICE, SLICE)],
                    part_hbm.at[cid, pl.ds(sid * SLICE, SLICE)])


def _combine_body(p_ref, o_ref):
    o_ref[...] = jnp.exp(p_ref[0, :] + p_ref[1, :])


@jax.jit
def _segment_prod(x, y):
    mesh = plsc.VectorSubcoreMesh(core_axis_name="c", subcore_axis_name="s")
    partials = pl.kernel(
        _sc_body,
        out_type=jax.ShapeDtypeStruct((NC, SEG_PAD), jnp.float32),
        mesh=mesh,
        scratch_types=[
            pltpu.VMEM((NSLOT, 1, CHUNK_E), jnp.float32),
            pltpu.VMEM((NSLOT, 1, CHUNK_E), jnp.int32),
            pltpu.VMEM((SLICE,), jnp.float32),
            pltpu.VMEM((1, LANES), jnp.float32),
            pltpu.VMEM((1, LANES), jnp.int32),
            pltpu.VMEM_SHARED((SEG_PAD,), jnp.float32),
            pltpu.SemaphoreType.DMA,
            pltpu.SemaphoreType.DMA,
            pltpu.SemaphoreType.DMA,
            pltpu.SemaphoreType.DMA,
            pltpu.SemaphoreType.DMA,
            pltpu.SemaphoreType.DMA,
        ],
        compiler_params=pltpu.CompilerParams(needs_layout_passes=False),
    )(x, y)

    combined = pl.pallas_call(
        _combine_body,
        in_specs=[pl.BlockSpec((NC, SEG_PAD), lambda: (0, 0))],
        out_specs=pl.BlockSpec((SEG_PAD,), lambda: (0,)),
        out_shape=jax.ShapeDtypeStruct((SEG_PAD,), jnp.float32),
    )(partials)
    return combined[:N_SEG]


def kernel(x, y, z):
    del z
    return _segment_prod(x, y)
